# Initial kernel scaffold; baseline (speedup 1.0000x reference)
import functools
import jax
import jax.numpy as jnp
from jax import lax
from jax.experimental import pallas as pl
from jax.experimental.pallas import tpu as pltpu
from jax.experimental.pallas import tpu_sc as plsc

B, L = 1024, 200
WORD_DIM, POS_DIM = 128, 32
OUT_DIM = WORD_DIM + 2 * POS_DIM
N = B * L
C = 128
NUM_CHUNKS = N // C
NUM_WORKERS = 32
CHUNKS_PER_WORKER = NUM_CHUNKS // NUM_WORKERS


def _sc_embed(widx, eidx, fidx, rdst, word_table, ent_table, fil_table):
    mesh = plsc.VectorSubcoreMesh(core_axis_name="c", subcore_axis_name="s")

    @functools.partial(
        pl.kernel,
        out_type=(
            jax.ShapeDtypeStruct((N, OUT_DIM), jnp.float32),
            jax.ShapeDtypeStruct((N, OUT_DIM), jnp.float32),
        ),
        mesh=mesh,
        scratch_types=[
            pltpu.VMEM((CHUNKS_PER_WORKER, C), jnp.int32),
            pltpu.VMEM((CHUNKS_PER_WORKER, C), jnp.int32),
            pltpu.VMEM((CHUNKS_PER_WORKER, C), jnp.int32),
            pltpu.VMEM((CHUNKS_PER_WORKER, C), jnp.int32),
            pltpu.VMEM((C, OUT_DIM), jnp.float32),
            pltpu.SemaphoreType.DMA,
            pltpu.SemaphoreType.DMA,
        ],
    )
    def kern(widx_hbm, eidx_hbm, fidx_hbm, rdst_hbm,
             wtab_hbm, etab_hbm, ftab_hbm,
             out_hbm, rout_hbm,
             widx_v, eidx_v, fidx_v, rdst_v, cbuf, gsem, wsem):
        wid = lax.axis_index("s") * 2 + lax.axis_index("c")
        row0 = wid * CHUNKS_PER_WORKER

        pltpu.sync_copy(widx_hbm.at[pl.ds(row0, CHUNKS_PER_WORKER)], widx_v)
        pltpu.sync_copy(eidx_hbm.at[pl.ds(row0, CHUNKS_PER_WORKER)], eidx_v)
        pltpu.sync_copy(fidx_hbm.at[pl.ds(row0, CHUNKS_PER_WORKER)], fidx_v)
        pltpu.sync_copy(rdst_hbm.at[pl.ds(row0, CHUNKS_PER_WORKER)], rdst_v)

        def body(j, carry):
            base = (row0 + j) * C
            gw = pltpu.async_copy(
                wtab_hbm.at[widx_v.at[j]], cbuf.at[:, pl.ds(0, WORD_DIM)], gsem)
            ge = pltpu.async_copy(
                etab_hbm.at[eidx_v.at[j]],
                cbuf.at[:, pl.ds(WORD_DIM, POS_DIM)], gsem)
            gf = pltpu.async_copy(
                ftab_hbm.at[fidx_v.at[j]],
                cbuf.at[:, pl.ds(WORD_DIM + POS_DIM, POS_DIM)], gsem)
            gw.wait()
            ge.wait()
            gf.wait()
            w1 = pltpu.async_copy(cbuf, out_hbm.at[pl.ds(base, C)], wsem)
            w2 = pltpu.async_copy(cbuf, rout_hbm.at[rdst_v.at[j]], wsem)
            w1.wait()
            w2.wait()
            return carry

        lax.fori_loop(0, CHUNKS_PER_WORKER, body, 0)

    return kern(widx, eidx, fidx, rdst, word_table, ent_table, fil_table)


def kernel(sentence, entity_position, filler_position,
           word_table, entity_pos_table, filler_pos_table):
    widx = sentence.astype(jnp.int32).reshape(NUM_CHUNKS, C)
    eidx = entity_position.astype(jnp.int32).reshape(NUM_CHUNKS, C)
    fidx = filler_position.astype(jnp.int32).reshape(NUM_CHUNKS, C)

    flat = lax.iota(jnp.int32, N)
    b = flat // L
    t = flat - b * L
    rdst = (b * L + (L - 1) - t).reshape(NUM_CHUNKS, C)

    out, rout = _sc_embed(widx, eidx, fidx, rdst,
                          word_table, entity_pos_table, filler_pos_table)
    return (out.reshape(B, L, OUT_DIM), rout.reshape(B, L, OUT_DIM))


# speedup vs baseline: 3.5975x; 3.5975x over previous
import functools
import jax
import jax.numpy as jnp
from jax import lax
from jax.experimental import pallas as pl
from jax.experimental.pallas import tpu as pltpu
from jax.experimental.pallas import tpu_sc as plsc

B, L = 1024, 200
WORD_DIM, POS_DIM = 128, 32
OUT_DIM = WORD_DIM + 2 * POS_DIM
POS_TAB = 512 * POS_DIM
N = B * L
C = 128
NUM_WORKERS = 32
TPW = N // NUM_WORKERS
CPW = TPW // C


def _sc_embed(widx, eidx, fidx, widx_r, eidx_r, fidx_r,
              word_table, etab_flat, ftab_flat):
    mesh = plsc.VectorSubcoreMesh(core_axis_name="c", subcore_axis_name="s")

    @functools.partial(
        pl.kernel,
        out_type=(
            jax.ShapeDtypeStruct((N, OUT_DIM), jnp.float32),
            jax.ShapeDtypeStruct((N, OUT_DIM), jnp.float32),
        ),
        mesh=mesh,
        scratch_types=[
            pltpu.VMEM((C,), jnp.int32),
            pltpu.VMEM((C,), jnp.int32),
            pltpu.VMEM((C,), jnp.int32),
            pltpu.VMEM((C,), jnp.int32),
            pltpu.VMEM((C,), jnp.int32),
            pltpu.VMEM((C,), jnp.int32),
            pltpu.VMEM((POS_TAB,), jnp.float32),
            pltpu.VMEM((POS_TAB,), jnp.float32),
            pltpu.VMEM((C, OUT_DIM), jnp.float32),
            pltpu.VMEM((C, OUT_DIM), jnp.float32),
            pltpu.SemaphoreType.DMA,
            pltpu.SemaphoreType.DMA,
        ],
    )
    def kern(widx_hbm, eidx_hbm, fidx_hbm, widxr_hbm, eidxr_hbm, fidxr_hbm,
             wtab_hbm, etab_hbm, ftab_hbm,
             out_hbm, rout_hbm,
             widx_v, eidx_v, fidx_v, widxr_v, eidxr_v, fidxr_v,
             etab_v, ftab_v, cbuf_f, cbuf_r, gsem, wsem):
        wid = lax.axis_index("s") * 2 + lax.axis_index("c")
        tok0 = wid * TPW

        pltpu.sync_copy(etab_hbm, etab_v)
        pltpu.sync_copy(ftab_hbm, ftab_v)

        def assemble(cbuf, e_v, f_v):
            def asm_body(g, carry):
                ev = e_v[pl.ds(g * 16, 16)] * POS_DIM
                fv = f_v[pl.ds(g * 16, 16)] * POS_DIM
                for r in range(16):
                    row = g * 16 + r
                    se = ev[r]
                    sf = fv[r]
                    cbuf[row, pl.ds(WORD_DIM, 16)] = etab_v[pl.ds(se, 16)]
                    cbuf[row, pl.ds(WORD_DIM + 16, 16)] = (
                        etab_v[pl.ds(se + 16, 16)])
                    cbuf[row, pl.ds(WORD_DIM + 32, 16)] = (
                        ftab_v[pl.ds(sf, 16)])
                    cbuf[row, pl.ds(WORD_DIM + 48, 16)] = (
                        ftab_v[pl.ds(sf + 16, 16)])
                return carry
            lax.fori_loop(0, C // 16, asm_body, 0)

        def body(j, carry):
            base = tok0 + j * C
            i1 = pltpu.async_copy(widx_hbm.at[pl.ds(base, C)], widx_v, gsem)
            i2 = pltpu.async_copy(eidx_hbm.at[pl.ds(base, C)], eidx_v, gsem)
            i3 = pltpu.async_copy(fidx_hbm.at[pl.ds(base, C)], fidx_v, gsem)
            i4 = pltpu.async_copy(widxr_hbm.at[pl.ds(base, C)], widxr_v, gsem)
            i5 = pltpu.async_copy(eidxr_hbm.at[pl.ds(base, C)], eidxr_v, gsem)
            i6 = pltpu.async_copy(fidxr_hbm.at[pl.ds(base, C)], fidxr_v, gsem)
            i1.wait()
            i2.wait()
            i3.wait()
            i4.wait()
            i5.wait()
            i6.wait()
            gwf = pltpu.async_copy(
                wtab_hbm.at[widx_v],
                cbuf_f.at[:, pl.ds(0, WORD_DIM)], gsem)
            gwr = pltpu.async_copy(
                wtab_hbm.at[widxr_v],
                cbuf_r.at[:, pl.ds(0, WORD_DIM)], gsem)
            assemble(cbuf_f, eidx_v, fidx_v)
            assemble(cbuf_r, eidxr_v, fidxr_v)
            gwf.wait()
            gwr.wait()
            w1 = pltpu.async_copy(cbuf_f, out_hbm.at[pl.ds(base, C)], wsem)
            w2 = pltpu.async_copy(cbuf_r, rout_hbm.at[pl.ds(base, C)], wsem)
            w1.wait()
            w2.wait()
            return carry

        lax.fori_loop(0, CPW, body, 0)

    return kern(widx, eidx, fidx, widx_r, eidx_r, fidx_r,
                word_table, etab_flat, ftab_flat)


def kernel(sentence, entity_position, filler_position,
           word_table, entity_pos_table, filler_pos_table):
    sentence = sentence.astype(jnp.int32)
    entity_position = entity_position.astype(jnp.int32)
    filler_position = filler_position.astype(jnp.int32)

    widx = sentence.reshape(N)
    eidx = entity_position.reshape(N)
    fidx = filler_position.reshape(N)
    widx_r = jnp.flip(sentence, axis=1).reshape(N)
    eidx_r = jnp.flip(entity_position, axis=1).reshape(N)
    fidx_r = jnp.flip(filler_position, axis=1).reshape(N)

    out, rout = _sc_embed(
        widx, eidx, fidx, widx_r, eidx_r, fidx_r,
        word_table,
        entity_pos_table.reshape(POS_TAB),
        filler_pos_table.reshape(POS_TAB))
    return (out.reshape(B, L, OUT_DIM), rout.reshape(B, L, OUT_DIM))
